# baseline (speedup 1.0000x reference)
import functools

import jax
import jax.numpy as jnp
from jax import lax
from jax.experimental import pallas as pl
from jax.experimental.pallas import tpu as pltpu
from jax.experimental.pallas import tpu_sc as plsc

K = 1024
D = 64
BETA = 0.25
B = 16
HW = 1024
N = B * HW

_SC_INFO = plsc.get_sparse_core_info()
_NW = _SC_INFO.num_cores * _SC_INFO.num_subcores
_B_PER_W = N // _NW


def _vq_kernel(x_ref, e_ref, idx_ref, stats_ref, esq_sc, iota_sc, counts_acc,
               loss_acc):
    b = pl.program_id(0)
    e = e_ref[...]

    @pl.when(b == 0)
    def _init():
        esq_sc[...] = jnp.sum(e * e, axis=1)[None, :]
        iota_sc[...] = jax.lax.broadcasted_iota(
            jnp.int32, (1, K), 1).astype(jnp.float32)
        counts_acc[...] = jnp.zeros_like(counts_acc)
        loss_acc[...] = jnp.zeros_like(loss_acc)

    ze = jnp.transpose(x_ref[0])
    xsq = jnp.sum(ze * ze, axis=1)
    scores2 = jax.lax.dot_general(
        ze, e + e, (((1,), (1,)), ((), ())),
        preferred_element_type=jnp.float32)
    dist = (xsq[:, None] + esq_sc[...]) - scores2

    mind = jnp.min(dist, axis=1)
    loss_acc[...] += mind
    masked = jnp.where(dist == mind[:, None], iota_sc[...], jnp.float32(K))
    idx_f = jnp.min(masked, axis=1)
    idx_ref[0, 0] = idx_f.astype(jnp.int32)

    onehot = (masked == idx_f[:, None]).astype(jnp.float32)
    counts_acc[...] += jax.lax.dot_general(
        jnp.ones((1, HW), jnp.float32), onehot, (((1,), (0,)), ((), ())),
        preferred_element_type=jnp.float32)

    @pl.when(b == B - 1)
    def _finalize():
        counts = counts_acc[0]
        prob = counts * (1.0 / N)
        entropy_bits = -jnp.sum(prob * jnp.log2(prob + 1e-10))
        est_words = jnp.exp2(entropy_bits)
        e_latent = jnp.sum(loss_acc[...]) * (1.0 / (N * D))
        stats_ref[0, 0] = (1.0 + BETA) * e_latent
        stats_ref[0, 1] = e_latent
        stats_ref[0, 2] = est_words


def _sc_gather_body(table_hbm, idx_hbm, out_hbm, idx_v, rows_v, sem):
    wid = lax.axis_index("s") * _SC_INFO.num_cores + lax.axis_index("c")
    base = wid * _B_PER_W
    pltpu.sync_copy(idx_hbm.at[pl.ds(base, _B_PER_W)], idx_v)
    pltpu.async_copy(table_hbm.at[idx_v], rows_v, sem).wait()
    pltpu.sync_copy(rows_v, out_hbm.at[pl.ds(base, _B_PER_W)])


def _transpose_kernel(rows_ref, zq_ref):
    zq_ref[0] = jnp.transpose(rows_ref[0][:, :D])


@jax.jit
def kernel(inputs, E_weight):
    x3 = inputs.reshape(B, D, HW)
    idx3, stats = pl.pallas_call(
        _vq_kernel,
        grid=(B,),
        in_specs=[
            pl.BlockSpec((1, D, HW), lambda b: (b, 0, 0)),
            pl.BlockSpec((K, D), lambda b: (0, 0)),
        ],
        out_specs=[
            pl.BlockSpec((1, 1, HW), lambda b: (b, 0, 0)),
            pl.BlockSpec(memory_space=pltpu.SMEM),
        ],
        out_shape=[
            jax.ShapeDtypeStruct((B, 1, HW), jnp.int32),
            jax.ShapeDtypeStruct((1, 4), jnp.float32),
        ],
        scratch_shapes=[
            pltpu.VMEM((1, K), jnp.float32),
            pltpu.VMEM((1, K), jnp.float32),
            pltpu.VMEM((1, K), jnp.float32),
            pltpu.VMEM((HW,), jnp.float32),
        ],
    )(x3, E_weight)

    idx_flat = idx3.reshape(N)
    e_pad = jnp.concatenate(
        [E_weight, jnp.zeros((K, 128 - D), jnp.float32)], axis=1)
    gather = functools.partial(
        pl.kernel,
        mesh=plsc.VectorSubcoreMesh(core_axis_name="c", subcore_axis_name="s"),
        out_type=jax.ShapeDtypeStruct((N, 128), jnp.float32),
        scratch_types=[
            pltpu.VMEM((_B_PER_W,), jnp.int32),
            pltpu.VMEM((_B_PER_W, 128), jnp.float32),
            pltpu.SemaphoreType.DMA,
        ],
    )(_sc_gather_body)
    zq_rows = gather(e_pad, idx_flat)

    zq3 = pl.pallas_call(
        _transpose_kernel,
        grid=(B,),
        in_specs=[pl.BlockSpec((1, HW, 128), lambda b: (b, 0, 0))],
        out_specs=pl.BlockSpec((1, D, HW), lambda b: (b, 0, 0)),
        out_shape=jax.ShapeDtypeStruct((B, D, HW), jnp.float32),
    )(zq_rows.reshape(B, HW, 128))

    zq = zq3.reshape(B, D, 32, 32)
    e_and_q = stats[0, 0]
    e_latent = stats[0, 1]
    est_words = stats[0, 2]
    return (e_and_q, zq, e_latent, e_latent, est_words)


# speedup vs baseline: 1.4874x; 1.4874x over previous
import jax
import jax.numpy as jnp
from jax.experimental import pallas as pl
from jax.experimental.pallas import tpu as pltpu

K = 1024
D = 64
BETA = 0.25
B = 16
HW = 1024
N = B * HW


HB = HW // 2


def _vq_kernel(x_ref, e_ref, zq_ref, stats_ref, esq_sc, iota_sc, counts_acc,
               loss_acc):
    b = pl.program_id(0)
    h = pl.program_id(1)
    e = e_ref[...]

    @pl.when(jnp.logical_and(b == 0, h == 0))
    def _init():
        esq_sc[...] = jnp.sum(e * e, axis=1)[None, :]
        iota_sc[...] = jax.lax.broadcasted_iota(
            jnp.int32, (1, K), 1).astype(jnp.float32)
        counts_acc[...] = jnp.zeros_like(counts_acc)
        loss_acc[...] = jnp.zeros_like(loss_acc)

    ze = jnp.transpose(x_ref[0])
    xsq = jnp.sum(ze * ze, axis=1)
    scores2 = jax.lax.dot_general(
        ze, e + e, (((1,), (1,)), ((), ())),
        preferred_element_type=jnp.float32)
    dist = (xsq[:, None] + esq_sc[...]) - scores2

    mind = jnp.min(dist, axis=1)
    loss_acc[...] += mind
    masked = jnp.where(dist == mind[:, None], iota_sc[...], jnp.float32(K))
    idx_f = jnp.min(masked, axis=1)

    onehot = (masked == idx_f[:, None]).astype(jnp.float32)
    zq_ref[0] = jax.lax.dot_general(
        e, onehot, (((0,), (1,)), ((), ())),
        preferred_element_type=jnp.float32)

    counts_acc[...] += jax.lax.dot_general(
        jnp.ones((1, HB), jnp.float32), onehot, (((1,), (0,)), ((), ())),
        preferred_element_type=jnp.float32)

    @pl.when(jnp.logical_and(b == B - 1, h == 1))
    def _finalize():
        counts = counts_acc[0]
        prob = counts * (1.0 / N)
        entropy_bits = -jnp.sum(prob * jnp.log2(prob + 1e-10))
        est_words = jnp.exp2(entropy_bits)
        e_latent = jnp.sum(loss_acc[...]) * (1.0 / (N * D))
        stats_ref[0, 0] = (1.0 + BETA) * e_latent
        stats_ref[0, 1] = e_latent
        stats_ref[0, 2] = est_words


@jax.jit
def kernel(inputs, E_weight):
    x3 = inputs.reshape(B, D, HW)
    zq3, stats = pl.pallas_call(
        _vq_kernel,
        grid=(B, 2),
        in_specs=[
            pl.BlockSpec((1, D, HB), lambda b, h: (b, 0, h)),
            pl.BlockSpec((K, D), lambda b, h: (0, 0)),
        ],
        out_specs=[
            pl.BlockSpec((1, D, HB), lambda b, h: (b, 0, h)),
            pl.BlockSpec(memory_space=pltpu.SMEM),
        ],
        out_shape=[
            jax.ShapeDtypeStruct((B, D, HW), jnp.float32),
            jax.ShapeDtypeStruct((1, 4), jnp.float32),
        ],
        scratch_shapes=[
            pltpu.VMEM((1, K), jnp.float32),
            pltpu.VMEM((1, K), jnp.float32),
            pltpu.VMEM((1, K), jnp.float32),
            pltpu.VMEM((HB,), jnp.float32),
        ],
    )(x3, E_weight)
    zq = zq3.reshape(B, D, 32, 32)
    e_and_q = stats[0, 0]
    e_latent = stats[0, 1]
    est_words = stats[0, 2]
    return (e_and_q, zq, e_latent, e_latent, est_words)


# speedup vs baseline: 1.7223x; 1.1580x over previous
import jax
import jax.numpy as jnp
from jax.experimental import pallas as pl
from jax.experimental.pallas import tpu as pltpu

K = 1024
D = 64
BETA = 0.25
B = 16
HW = 1024
N = B * HW


def _vq_kernel(x_ref, e_ref, zq_ref, stats_ref, esq_sc, iota_sc, counts_acc,
               loss_acc):
    b = pl.program_id(0)
    e = e_ref[...]

    @pl.when(b == 0)
    def _init():
        esq_sc[...] = jnp.sum(e * e, axis=1)[None, :]
        iota_sc[...] = jax.lax.broadcasted_iota(
            jnp.int32, (1, K), 1).astype(jnp.float32)
        counts_acc[...] = jnp.zeros_like(counts_acc)
        loss_acc[...] = jnp.zeros_like(loss_acc)

    ze = jnp.transpose(x_ref[...], (0, 2, 1)).reshape(2 * HW, D)
    xsq = jnp.sum(ze * ze, axis=1)
    scores2 = jax.lax.dot_general(
        ze, e + e, (((1,), (1,)), ((), ())),
        preferred_element_type=jnp.float32)
    dist = (xsq[:, None] + esq_sc[...]) - scores2

    mind = jnp.min(dist, axis=1)
    loss_acc[...] += mind
    masked = jnp.where(dist == mind[:, None], iota_sc[...], jnp.float32(K))
    idx_f = jnp.min(masked, axis=1)

    onehot = (masked == idx_f[:, None]).astype(jnp.float32)
    zq_ref[0] = jax.lax.dot_general(
        e, onehot[:HW], (((0,), (1,)), ((), ())),
        preferred_element_type=jnp.float32)
    zq_ref[1] = jax.lax.dot_general(
        e, onehot[HW:], (((0,), (1,)), ((), ())),
        preferred_element_type=jnp.float32)

    counts_acc[...] += jax.lax.dot_general(
        jnp.ones((1, 2 * HW), jnp.float32), onehot, (((1,), (0,)), ((), ())),
        preferred_element_type=jnp.float32)

    @pl.when(b == B // 2 - 1)
    def _finalize():
        counts = counts_acc[0]
        prob = counts * (1.0 / N)
        entropy_bits = -jnp.sum(prob * jnp.log2(prob + 1e-10))
        est_words = jnp.exp2(entropy_bits)
        e_latent = jnp.sum(loss_acc[...]) * (1.0 / (N * D))
        stats_ref[0, 0] = (1.0 + BETA) * e_latent
        stats_ref[0, 1] = e_latent
        stats_ref[0, 2] = est_words


@jax.jit
def kernel(inputs, E_weight):
    x3 = inputs.reshape(B, D, HW)
    zq3, stats = pl.pallas_call(
        _vq_kernel,
        grid=(B // 2,),
        in_specs=[
            pl.BlockSpec((2, D, HW), lambda b: (b, 0, 0)),
            pl.BlockSpec((K, D), lambda b: (0, 0)),
        ],
        out_specs=[
            pl.BlockSpec((2, D, HW), lambda b: (b, 0, 0)),
            pl.BlockSpec(memory_space=pltpu.SMEM),
        ],
        out_shape=[
            jax.ShapeDtypeStruct((B, D, HW), jnp.float32),
            jax.ShapeDtypeStruct((1, 4), jnp.float32),
        ],
        scratch_shapes=[
            pltpu.VMEM((1, K), jnp.float32),
            pltpu.VMEM((1, K), jnp.float32),
            pltpu.VMEM((1, K), jnp.float32),
            pltpu.VMEM((2 * HW,), jnp.float32),
        ],
    )(x3, E_weight)
    zq = zq3.reshape(B, D, 32, 32)
    e_and_q = stats[0, 0]
    e_latent = stats[0, 1]
    est_words = stats[0, 2]
    return (e_and_q, zq, e_latent, e_latent, est_words)


# speedup vs baseline: 1.7623x; 1.0232x over previous
import jax
import jax.numpy as jnp
from jax.experimental import pallas as pl
from jax.experimental.pallas import tpu as pltpu

K = 1024
D = 64
BETA = 0.25
B = 16
HW = 1024
N = B * HW


def _vq_kernel(x_ref, e_ref, zq_ref, stats_ref, esq_sc, iota_sc, counts_acc,
               loss_acc):
    b = pl.program_id(0)
    e = e_ref[...]

    @pl.when(b == 0)
    def _init():
        esq_sc[...] = jnp.sum(e * e, axis=1)[None, :]
        iota_sc[...] = jax.lax.broadcasted_iota(
            jnp.int32, (1, K), 1).astype(jnp.float32)
        counts_acc[...] = jnp.zeros_like(counts_acc)
        loss_acc[...] = jnp.zeros_like(loss_acc)

    ze = jnp.transpose(x_ref[...], (0, 2, 1)).reshape(4 * HW, D)
    xsq = jnp.sum(ze * ze, axis=1)
    scores2 = jax.lax.dot_general(
        ze, e + e, (((1,), (1,)), ((), ())),
        preferred_element_type=jnp.float32)
    dist = (xsq[:, None] + esq_sc[...]) - scores2

    mind = jnp.min(dist, axis=1)
    loss_acc[...] += mind
    masked = jnp.where(dist == mind[:, None], iota_sc[...], jnp.float32(K))
    idx_f = jnp.min(masked, axis=1)

    onehot = (masked == idx_f[:, None]).astype(jnp.float32)
    zq_ref[0] = jax.lax.dot_general(
        e, onehot[:HW], (((0,), (1,)), ((), ())),
        preferred_element_type=jnp.float32)
    zq_ref[1] = jax.lax.dot_general(
        e, onehot[HW:2 * HW], (((0,), (1,)), ((), ())),
        preferred_element_type=jnp.float32)
    zq_ref[2] = jax.lax.dot_general(
        e, onehot[2 * HW:3 * HW], (((0,), (1,)), ((), ())),
        preferred_element_type=jnp.float32)
    zq_ref[3] = jax.lax.dot_general(
        e, onehot[3 * HW:], (((0,), (1,)), ((), ())),
        preferred_element_type=jnp.float32)

    counts_acc[...] += jax.lax.dot_general(
        jnp.ones((1, 4 * HW), jnp.float32), onehot, (((1,), (0,)), ((), ())),
        preferred_element_type=jnp.float32)

    @pl.when(b == B // 4 - 1)
    def _finalize():
        counts = counts_acc[0]
        prob = counts * (1.0 / N)
        entropy_bits = -jnp.sum(prob * jnp.log2(prob + 1e-10))
        est_words = jnp.exp2(entropy_bits)
        e_latent = jnp.sum(loss_acc[...]) * (1.0 / (N * D))
        stats_ref[0, 0] = (1.0 + BETA) * e_latent
        stats_ref[0, 1] = e_latent
        stats_ref[0, 2] = est_words


@jax.jit
def kernel(inputs, E_weight):
    x3 = inputs.reshape(B, D, HW)
    zq3, stats = pl.pallas_call(
        _vq_kernel,
        grid=(B // 4,),
        in_specs=[
            pl.BlockSpec((4, D, HW), lambda b: (b, 0, 0)),
            pl.BlockSpec((K, D), lambda b: (0, 0)),
        ],
        out_specs=[
            pl.BlockSpec((4, D, HW), lambda b: (b, 0, 0)),
            pl.BlockSpec(memory_space=pltpu.SMEM),
        ],
        out_shape=[
            jax.ShapeDtypeStruct((B, D, HW), jnp.float32),
            jax.ShapeDtypeStruct((1, 4), jnp.float32),
        ],
        scratch_shapes=[
            pltpu.VMEM((1, K), jnp.float32),
            pltpu.VMEM((1, K), jnp.float32),
            pltpu.VMEM((1, K), jnp.float32),
            pltpu.VMEM((4 * HW,), jnp.float32),
        ],
    )(x3, E_weight)
    zq = zq3.reshape(B, D, 32, 32)
    e_and_q = stats[0, 0]
    e_latent = stats[0, 1]
    est_words = stats[0, 2]
    return (e_and_q, zq, e_latent, e_latent, est_words)
